# baseline (speedup 1.0000x reference)
import functools

import jax
import jax.numpy as jnp
from jax import lax
from jax.experimental import pallas as pl
from jax.experimental.pallas import tpu as pltpu
from jax.experimental.pallas import tpu_sc as plsc

_N_E = 1024
_E_DIM = 256
_BETA = 0.25
_B = 16
_HW = 1024
_ROWS = _B * _HW


def _vq_body(zf_ref, emb_ref, idx_ref, cnt_ref, loss_ref, perp_ref):
    i = pl.program_id(0)
    zb = zf_ref[...]
    em = emb_ref[...]

    rn = jnp.sum(zb * zb, axis=1, keepdims=True)
    en = jnp.sum(em * em, axis=1)
    mm = jax.lax.dot_general(
        zb, em, (((1,), (1,)), ((), ())),
        preferred_element_type=jnp.float32)
    d = (rn + en[None, :]) - 2.0 * mm

    dmin = jnp.min(d, axis=1, keepdims=True)
    kio = jax.lax.broadcasted_iota(jnp.int32, d.shape, 1)
    idxv = jnp.min(jnp.where(d == dmin, kio, jnp.int32(2**30)), axis=1)
    idx_ref[0, 0, :] = idxv

    oh = (kio == idxv[:, None]).astype(jnp.float32)

    @pl.when(i == 0)
    def _init():
        cnt_ref[...] = jnp.zeros_like(cnt_ref)
        loss_ref[...] = jnp.zeros_like(loss_ref)

    cnt_ref[0, :] += jnp.sum(oh, axis=0)
    loss_ref[...] += jnp.sum(dmin, keepdims=True)

    @pl.when(i == _B - 1)
    def _finalize():
        loss_ref[...] = loss_ref[...] * ((1.0 + _BETA) / (_ROWS * _E_DIM))
        cnt = cnt_ref[...]
        e_mean = cnt / jnp.sum(cnt, axis=1, keepdims=True)
        ent = jnp.sum(e_mean * jnp.log(e_mean + 1e-10), axis=1, keepdims=True)
        perp_ref[...] = jnp.exp(-ent)


def _vq_call(zf, emb):
    return pl.pallas_call(
        _vq_body,
        grid=(_B,),
        in_specs=[
            pl.BlockSpec((_HW, _E_DIM), lambda i: (i, 0)),
            pl.BlockSpec((_N_E, _E_DIM), lambda i: (0, 0)),
        ],
        out_specs=[
            pl.BlockSpec((1, 1, _N_E), lambda i: (i, 0, 0)),
            pl.BlockSpec((1, _N_E), lambda i: (0, 0)),
            pl.BlockSpec((1, 1), lambda i: (0, 0)),
            pl.BlockSpec((1, 1), lambda i: (0, 0)),
        ],
        out_shape=[
            jax.ShapeDtypeStruct((_B, 1, _HW), jnp.int32),
            jax.ShapeDtypeStruct((1, _N_E), jnp.float32),
            jax.ShapeDtypeStruct((1, 1), jnp.float32),
            jax.ShapeDtypeStruct((1, 1), jnp.float32),
        ],
        compiler_params=pltpu.CompilerParams(
            dimension_semantics=("arbitrary",),
        ),
    )(zf, emb)


_NW = 32
_RPW = _ROWS // _NW
_CHUNK = 256


@functools.partial(
    pl.kernel,
    mesh=plsc.VectorSubcoreMesh(core_axis_name="c", subcore_axis_name="s"),
    out_type=jax.ShapeDtypeStruct((_ROWS, _E_DIM), jnp.float32),
    scratch_types=[
        pltpu.VMEM((_CHUNK,), jnp.int32),
        pltpu.VMEM((_CHUNK, _E_DIM), jnp.float32),
        pltpu.SemaphoreType.DMA,
    ],
)
def _sc_gather(emb_hbm, idx_hbm, out_hbm, idx_v, rows_v, sem):
    wid = lax.axis_index("s") * 2 + lax.axis_index("c")
    base = wid * _RPW
    for j in range(_RPW // _CHUNK):
        off = base + j * _CHUNK
        pltpu.sync_copy(idx_hbm.at[pl.ds(off, _CHUNK)], idx_v)
        pltpu.async_copy(emb_hbm.at[idx_v], rows_v, sem).wait()
        pltpu.sync_copy(rows_v, out_hbm.at[pl.ds(off, _CHUNK)])


def kernel(z, emb):
    b, c, h, w = z.shape
    zf = jnp.transpose(z, (0, 2, 3, 1)).reshape(-1, c)
    idx3, cnt, loss, perp = _vq_call(zf, emb)
    idx_flat = idx3.reshape(_ROWS)
    zq = _sc_gather(emb, idx_flat)
    enc_idx = idx3.reshape(b, h, w)
    z_q_out = jnp.transpose(zq.reshape(b, h, w, c), (0, 3, 1, 2))
    return (loss.reshape(()), z_q_out, perp.reshape(()),
            cnt.reshape(_N_E), enc_idx)


# speedup vs baseline: 1.1696x; 1.1696x over previous
import jax
import jax.numpy as jnp
from jax.experimental import pallas as pl
from jax.experimental.pallas import tpu as pltpu

_N_E = 1024
_E_DIM = 256
_BETA = 0.25
_B = 16
_HW = 1024
_ROWS = _B * _HW
_BLK = 512
_NBLK = _ROWS // _BLK


def _vq_body(zf_ref, emb_ref, idx_ref, zq_ref, cnt_ref, loss_ref, perp_ref):
    i = pl.program_id(0)
    zb = zf_ref[...]
    em = emb_ref[...]

    rn = jnp.sum(zb * zb, axis=1, keepdims=True)
    en = jnp.sum(em * em, axis=1)
    mm = jax.lax.dot_general(
        zb, em, (((1,), (1,)), ((), ())),
        preferred_element_type=jnp.float32)
    d = (rn + en[None, :]) - 2.0 * mm

    dmin = jnp.min(d, axis=1, keepdims=True)
    kio = jax.lax.broadcasted_iota(jnp.int32, d.shape, 1)
    idxv = jnp.min(jnp.where(d == dmin, kio, jnp.int32(2**30)), axis=1)
    idx_ref[0, 0, :] = idxv

    oh = (kio == idxv[:, None]).astype(jnp.float32)
    zq = jax.lax.dot_general(
        oh, em, (((1,), (0,)), ((), ())),
        preferred_element_type=jnp.float32)
    zq_ref[...] = zb + (zq - zb)

    @pl.when(i == 0)
    def _init():
        cnt_ref[...] = jnp.zeros_like(cnt_ref)
        loss_ref[...] = jnp.zeros_like(loss_ref)

    cnt_ref[0, :] += jnp.sum(oh, axis=0)
    loss_ref[...] += jnp.sum(dmin, keepdims=True)

    @pl.when(i == _NBLK - 1)
    def _finalize():
        loss_ref[...] = loss_ref[...] * ((1.0 + _BETA) / (_ROWS * _E_DIM))
        cnt = cnt_ref[...]
        e_mean = cnt / jnp.sum(cnt, axis=1, keepdims=True)
        ent = jnp.sum(e_mean * jnp.log(e_mean + 1e-10), axis=1, keepdims=True)
        perp_ref[...] = jnp.exp(-ent)


def _vq_call(zf, emb):
    return pl.pallas_call(
        _vq_body,
        grid=(_NBLK,),
        in_specs=[
            pl.BlockSpec((_BLK, _E_DIM), lambda i: (i, 0)),
            pl.BlockSpec((_N_E, _E_DIM), lambda i: (0, 0)),
        ],
        out_specs=[
            pl.BlockSpec((1, 1, _BLK), lambda i: (i, 0, 0)),
            pl.BlockSpec((_BLK, _E_DIM), lambda i: (i, 0)),
            pl.BlockSpec((1, _N_E), lambda i: (0, 0)),
            pl.BlockSpec((1, 1), lambda i: (0, 0)),
            pl.BlockSpec((1, 1), lambda i: (0, 0)),
        ],
        out_shape=[
            jax.ShapeDtypeStruct((_NBLK, 1, _BLK), jnp.int32),
            jax.ShapeDtypeStruct((_ROWS, _E_DIM), jnp.float32),
            jax.ShapeDtypeStruct((1, _N_E), jnp.float32),
            jax.ShapeDtypeStruct((1, 1), jnp.float32),
            jax.ShapeDtypeStruct((1, 1), jnp.float32),
        ],
        compiler_params=pltpu.CompilerParams(
            dimension_semantics=("arbitrary",),
        ),
    )(zf, emb)


def kernel(z, emb):
    b, c, h, w = z.shape
    zf = jnp.transpose(z, (0, 2, 3, 1)).reshape(-1, c)
    idx3, zq, cnt, loss, perp = _vq_call(zf, emb)
    enc_idx = idx3.reshape(b, h, w)
    z_q_out = jnp.transpose(zq.reshape(b, h, w, c), (0, 3, 1, 2))
    return (loss.reshape(()), z_q_out, perp.reshape(()),
            cnt.reshape(_N_E), enc_idx)


# speedup vs baseline: 1.4301x; 1.2228x over previous
import jax
import jax.numpy as jnp
from jax.experimental import pallas as pl
from jax.experimental.pallas import tpu as pltpu

_N_E = 1024
_E_DIM = 256
_BETA = 0.25
_B = 16
_HW = 1024
_ROWS = _B * _HW
_BLK = 2048
_NBLK = _ROWS // _BLK


def _vq_body(zf_ref, emb_ref, idx_ref, zq_ref, cnt_ref, loss_ref, perp_ref):
    i = pl.program_id(0)
    zb = zf_ref[...]
    em = emb_ref[...]

    rn = jnp.sum(zb * zb, axis=1, keepdims=True)
    en = jnp.sum(em * em, axis=1)
    mm = jax.lax.dot_general(
        zb, em, (((1,), (1,)), ((), ())),
        preferred_element_type=jnp.float32)
    d = (rn + en[None, :]) - 2.0 * mm

    dmin = jnp.min(d, axis=1, keepdims=True)
    kio = jax.lax.broadcasted_iota(jnp.int32, d.shape, 1)
    idxv = jnp.min(jnp.where(d == dmin, kio, jnp.int32(2**30)), axis=1)
    idx_ref[0, 0, :] = idxv

    oh = (kio == idxv[:, None]).astype(jnp.float32)
    zq = jax.lax.dot_general(
        oh, em, (((1,), (0,)), ((), ())),
        preferred_element_type=jnp.float32)
    zq_ref[...] = zb + (zq - zb)

    @pl.when(i == 0)
    def _init():
        cnt_ref[...] = jnp.zeros_like(cnt_ref)
        loss_ref[...] = jnp.zeros_like(loss_ref)

    cnt_ref[0, :] += jnp.sum(oh, axis=0)
    loss_ref[...] += jnp.sum(dmin, keepdims=True)

    @pl.when(i == _NBLK - 1)
    def _finalize():
        loss_ref[...] = loss_ref[...] * ((1.0 + _BETA) / (_ROWS * _E_DIM))
        cnt = cnt_ref[...]
        e_mean = cnt / jnp.sum(cnt, axis=1, keepdims=True)
        ent = jnp.sum(e_mean * jnp.log(e_mean + 1e-10), axis=1, keepdims=True)
        perp_ref[...] = jnp.exp(-ent)


def _vq_call(zf, emb):
    return pl.pallas_call(
        _vq_body,
        grid=(_NBLK,),
        in_specs=[
            pl.BlockSpec((_BLK, _E_DIM), lambda i: (i, 0)),
            pl.BlockSpec((_N_E, _E_DIM), lambda i: (0, 0)),
        ],
        out_specs=[
            pl.BlockSpec((1, 1, _BLK), lambda i: (i, 0, 0)),
            pl.BlockSpec((_BLK, _E_DIM), lambda i: (i, 0)),
            pl.BlockSpec((1, _N_E), lambda i: (0, 0)),
            pl.BlockSpec((1, 1), lambda i: (0, 0)),
            pl.BlockSpec((1, 1), lambda i: (0, 0)),
        ],
        out_shape=[
            jax.ShapeDtypeStruct((_NBLK, 1, _BLK), jnp.int32),
            jax.ShapeDtypeStruct((_ROWS, _E_DIM), jnp.float32),
            jax.ShapeDtypeStruct((1, _N_E), jnp.float32),
            jax.ShapeDtypeStruct((1, 1), jnp.float32),
            jax.ShapeDtypeStruct((1, 1), jnp.float32),
        ],
        compiler_params=pltpu.CompilerParams(
            dimension_semantics=("arbitrary",),
        ),
    )(zf, emb)


def kernel(z, emb):
    b, c, h, w = z.shape
    zf = jnp.transpose(z, (0, 2, 3, 1)).reshape(-1, c)
    idx3, zq, cnt, loss, perp = _vq_call(zf, emb)
    enc_idx = idx3.reshape(b, h, w)
    z_q_out = jnp.transpose(zq.reshape(b, h, w, c), (0, 3, 1, 2))
    return (loss.reshape(()), z_q_out, perp.reshape(()),
            cnt.reshape(_N_E), enc_idx)


# speedup vs baseline: 1.4593x; 1.0204x over previous
import jax
import jax.numpy as jnp
from jax.experimental import pallas as pl
from jax.experimental.pallas import tpu as pltpu

_N_E = 1024
_E_DIM = 256
_BETA = 0.25
_B = 16
_HW = 1024
_ROWS = _B * _HW
_BLK = 4096
_NBLK = _ROWS // _BLK


def _vq_body(zf_ref, emb_ref, idx_ref, zq_ref, cnt_ref, loss_ref, perp_ref):
    i = pl.program_id(0)
    zb = zf_ref[...]
    em = emb_ref[...]

    rn = jnp.sum(zb * zb, axis=1, keepdims=True)
    en = jnp.sum(em * em, axis=1)
    mm = jax.lax.dot_general(
        zb, em, (((1,), (1,)), ((), ())),
        preferred_element_type=jnp.float32)
    d = (rn + en[None, :]) - 2.0 * mm

    dmin = jnp.min(d, axis=1, keepdims=True)
    kio = jax.lax.broadcasted_iota(jnp.int32, d.shape, 1)
    idxv = jnp.min(jnp.where(d == dmin, kio, jnp.int32(2**30)), axis=1)
    idx_ref[0, 0, :] = idxv

    oh = (kio == idxv[:, None]).astype(jnp.float32)
    zq = jax.lax.dot_general(
        oh, em, (((1,), (0,)), ((), ())),
        preferred_element_type=jnp.float32)
    zq_ref[...] = zb + (zq - zb)

    @pl.when(i == 0)
    def _init():
        cnt_ref[...] = jnp.zeros_like(cnt_ref)
        loss_ref[...] = jnp.zeros_like(loss_ref)

    cnt_ref[0, :] += jnp.sum(oh, axis=0)
    loss_ref[...] += jnp.sum(dmin, keepdims=True)

    @pl.when(i == _NBLK - 1)
    def _finalize():
        loss_ref[...] = loss_ref[...] * ((1.0 + _BETA) / (_ROWS * _E_DIM))
        cnt = cnt_ref[...]
        e_mean = cnt / jnp.sum(cnt, axis=1, keepdims=True)
        ent = jnp.sum(e_mean * jnp.log(e_mean + 1e-10), axis=1, keepdims=True)
        perp_ref[...] = jnp.exp(-ent)


def _vq_call(zf, emb):
    return pl.pallas_call(
        _vq_body,
        grid=(_NBLK,),
        in_specs=[
            pl.BlockSpec((_BLK, _E_DIM), lambda i: (i, 0)),
            pl.BlockSpec((_N_E, _E_DIM), lambda i: (0, 0)),
        ],
        out_specs=[
            pl.BlockSpec((1, 1, _BLK), lambda i: (i, 0, 0)),
            pl.BlockSpec((_BLK, _E_DIM), lambda i: (i, 0)),
            pl.BlockSpec((1, _N_E), lambda i: (0, 0)),
            pl.BlockSpec((1, 1), lambda i: (0, 0)),
            pl.BlockSpec((1, 1), lambda i: (0, 0)),
        ],
        out_shape=[
            jax.ShapeDtypeStruct((_NBLK, 1, _BLK), jnp.int32),
            jax.ShapeDtypeStruct((_ROWS, _E_DIM), jnp.float32),
            jax.ShapeDtypeStruct((1, _N_E), jnp.float32),
            jax.ShapeDtypeStruct((1, 1), jnp.float32),
            jax.ShapeDtypeStruct((1, 1), jnp.float32),
        ],
        compiler_params=pltpu.CompilerParams(
            dimension_semantics=("arbitrary",),
        ),
    )(zf, emb)


def kernel(z, emb):
    b, c, h, w = z.shape
    zf = jnp.transpose(z, (0, 2, 3, 1)).reshape(-1, c)
    idx3, zq, cnt, loss, perp = _vq_call(zf, emb)
    enc_idx = idx3.reshape(b, h, w)
    z_q_out = jnp.transpose(zq.reshape(b, h, w, c), (0, 3, 1, 2))
    return (loss.reshape(()), z_q_out, perp.reshape(()),
            cnt.reshape(_N_E), enc_idx)
